# baseline (device time: 82060 ns/iter reference)
import jax
import jax.numpy as jnp
from jax import lax
from jax.experimental import pallas as pl
from jax.experimental.pallas import tpu as pltpu

N_DEV = 16
S = 8
OFF = [j // 2 for j in range(S)]


def kernel(x, dy):
    m, d = x.shape
    _, f = dy.shape
    chunk = d // N_DEV
    fs = f // S
    nh = N_DEV - 1

    def body(x_ref, dy_ref, out_ref,
             xb_ref, partial_ref, send_ref, comm_ref,
             send_sems, recv_sems):
        my = lax.axis_index("i")
        right = lax.rem(my + 1, N_DEV)
        left = lax.rem(my + N_DEV - 1, N_DEV)

        barrier_sem = pltpu.get_barrier_semaphore()
        for nbr in (left, right):
            pl.semaphore_signal(
                barrier_sem, inc=1,
                device_id=(nbr,), device_id_type=pl.DeviceIdType.MESH,
            )
        pl.semaphore_wait(barrier_sem, 2)

        xb_ref[...] = x_ref[...].astype(jnp.bfloat16)

        def fcols(j):
            return slice(j * fs, (j + 1) * fs)

        def c_send(j, s):
            if j % 2 == 0:
                return lax.rem(my + 2 * N_DEV - 1 - s, N_DEV)
            return lax.rem(my + 1 + s, N_DEV)

        def prow(c):
            return pl.ds(c * chunk, chunk)

        def make_rdma(j, s):
            return pltpu.make_async_remote_copy(
                src_ref=send_ref.at[j, s % 2],
                dst_ref=comm_ref.at[j, s],
                send_sem=send_sems.at[j, s % 2],
                recv_sem=recv_sems.at[j, s],
                device_id=(right if j % 2 == 0 else left,),
                device_id_type=pl.DeviceIdType.MESH,
            )

        def gemm_and_first_send(j):
            partial_ref[:, fcols(j)] = lax.dot_general(
                xb_ref[...], dy_ref[:, fcols(j)].astype(jnp.bfloat16),
                dimension_numbers=(((0,), (0,)), ((), ())),
                preferred_element_type=jnp.float32,
            ).astype(jnp.bfloat16)
            send_ref[j, 0] = partial_ref[prow(c_send(j, 0)), fcols(j)]
            make_rdma(j, 0).start()

        def process(j, s):
            make_rdma(j, s).wait_recv()
            if s < nh - 1:
                c = c_send(j, s + 1)
                if s >= 1:
                    make_rdma(j, s - 1).wait_send()
                send_ref[j, (s + 1) % 2] = (
                    comm_ref[j, s] + partial_ref[prow(c), fcols(j)])
                make_rdma(j, s + 1).start()
            else:
                out_ref[:, fcols(j)] = (
                    comm_ref[j, s].astype(jnp.float32)
                    + partial_ref[prow(my), fcols(j)].astype(jnp.float32))

        for j in range(S):
            if OFF[j] == 0:
                gemm_and_first_send(j)

        for t in range(nh + OFF[-1]):
            for j in range(S):
                if OFF[j] == t + 1:
                    gemm_and_first_send(j)
            for j in range(S):
                s = t - OFF[j]
                if 0 <= s < nh:
                    process(j, s)

        for j in range(S):
            for s in (nh - 2, nh - 1):
                make_rdma(j, s).wait_send()

    return pl.pallas_call(
        body,
        out_shape=jax.ShapeDtypeStruct((chunk, f), jnp.float32),
        in_specs=[
            pl.BlockSpec(memory_space=pltpu.VMEM),
            pl.BlockSpec(memory_space=pltpu.VMEM),
        ],
        out_specs=pl.BlockSpec(memory_space=pltpu.VMEM),
        scratch_shapes=[
            pltpu.VMEM((m, d), jnp.bfloat16),
            pltpu.VMEM((d, f), jnp.bfloat16),
            pltpu.VMEM((S, 2, chunk, fs), jnp.bfloat16),
            pltpu.VMEM((S, nh, chunk, fs), jnp.bfloat16),
            pltpu.SemaphoreType.DMA((S, 2)),
            pltpu.SemaphoreType.DMA((S, nh)),
        ],
        compiler_params=pltpu.CompilerParams(collective_id=0),
    )(x, dy)


# device time: 68517 ns/iter; 1.1977x vs baseline; 1.1977x over previous
import jax
import jax.numpy as jnp
from jax import lax
from jax.experimental import pallas as pl
from jax.experimental.pallas import tpu as pltpu

N_DEV = 16
NP = 4
S = 4
OFF = [0, 0, 1, 1]


def kernel(x, dy):
    m, d = x.shape
    _, f = dy.shape
    chunk = d // N_DEV
    fs = f // S
    ar = NP * chunk

    def body(x_ref, dy_ref, out_ref,
             xb_ref, arr_ref, sa_ref, ca_ref, ra_ref, rb_ref,
             sb1_ref, cb1_ref, sb2_ref, cb2_ref,
             sa_send_sems, sa_recv_sems,
             b1_send_sems, b1_recv_sems, b2_send_sems, b2_recv_sems):
        my = lax.axis_index("i")
        z = lax.div(my, NP)
        q = lax.rem(my, NP)
        plane_r = NP * z + lax.rem(q + 1, NP)
        plane_l = NP * z + lax.rem(q + NP - 1, NP)
        zl = lax.rem(z, 2)
        zh = lax.div(z, 2)
        dev_b1 = my + NP * (1 - 2 * zl)
        dev_b2 = my + 2 * NP * (1 - 2 * zh)

        barrier_sem = pltpu.get_barrier_semaphore()
        for nbr in (plane_l, plane_r, dev_b1, dev_b2):
            pl.semaphore_signal(
                barrier_sem, inc=1,
                device_id=(nbr,), device_id_type=pl.DeviceIdType.MESH,
            )
        pl.semaphore_wait(barrier_sem, 4)

        xb_ref[...] = x_ref[...].astype(jnp.bfloat16)

        def fcols(j):
            return slice(j * fs, (j + 1) * fs)

        def a_send_idx(j, s):
            if j % 2 == 0:
                return lax.rem(q + 2 * NP - 1 - s, NP)
            return lax.rem(q + 1 + s, NP)

        def make_a_rdma(j, s):
            return pltpu.make_async_remote_copy(
                src_ref=sa_ref.at[j, s % 2],
                dst_ref=ca_ref.at[j, s],
                send_sem=sa_send_sems.at[j, s % 2],
                recv_sem=sa_recv_sems.at[j, s],
                device_id=(plane_r if j % 2 == 0 else plane_l,),
                device_id_type=pl.DeviceIdType.MESH,
            )

        def make_b1_rdma(j):
            return pltpu.make_async_remote_copy(
                src_ref=sb1_ref.at[j], dst_ref=cb1_ref.at[j],
                send_sem=b1_send_sems.at[j], recv_sem=b1_recv_sems.at[j],
                device_id=(dev_b1,), device_id_type=pl.DeviceIdType.MESH,
            )

        def make_b2_rdma(j):
            return pltpu.make_async_remote_copy(
                src_ref=sb2_ref.at[j], dst_ref=cb2_ref.at[j],
                send_sem=b2_send_sems.at[j], recv_sem=b2_recv_sems.at[j],
                device_id=(dev_b2,), device_id_type=pl.DeviceIdType.MESH,
            )

        def gemm_and_first_send(j):
            val = lax.dot_general(
                xb_ref[...], dy_ref[:, fcols(j)].astype(jnp.bfloat16),
                dimension_numbers=(((0,), (0,)), ((), ())),
                preferred_element_type=jnp.float32,
            ).astype(jnp.bfloat16)
            for qb in range(NP):
                for zb in range(NP):
                    arr_ref[qb, zb * chunk:(zb + 1) * chunk, fcols(j)] = (
                        val[ar * zb + chunk * qb:ar * zb + chunk * (qb + 1), :])
            sa_ref[j, 0] = arr_ref[a_send_idx(j, 0), :, fcols(j)]
            make_a_rdma(j, 0).start()

        def process_a(j, s):
            make_a_rdma(j, s).wait_recv()
            a = a_send_idx(j, s + 1)
            acc = ca_ref[j, s] + arr_ref[a, :, fcols(j)]
            if s < 2:
                if s >= 1:
                    make_a_rdma(j, s - 1).wait_send()
                sa_ref[j, (s + 1) % 2] = acc
                make_a_rdma(j, s + 1).start()
            else:
                ra_ref[j] = acc
                lp = 1 - zl
                sb1_ref[j, 0:chunk] = ra_ref[j, pl.ds(chunk * lp, chunk), :]
                sb1_ref[j, chunk:2 * chunk] = (
                    ra_ref[j, pl.ds(chunk * (lp + 2), chunk), :])
                make_b1_rdma(j).start()

        def process_b1(j):
            make_b1_rdma(j).wait_recv()
            rb_ref[j, 0:chunk] = (
                cb1_ref[j, 0:chunk] + ra_ref[j, pl.ds(chunk * zl, chunk), :])
            rb_ref[j, chunk:2 * chunk] = (
                cb1_ref[j, chunk:2 * chunk]
                + ra_ref[j, pl.ds(chunk * (zl + 2), chunk), :])
            sb2_ref[j] = rb_ref[j, pl.ds(chunk * (1 - zh), chunk), :]
            make_b2_rdma(j).start()

        def process_b2(j):
            make_b2_rdma(j).wait_recv()
            out_ref[:, fcols(j)] = (
                cb2_ref[j].astype(jnp.float32)
                + rb_ref[j, pl.ds(chunk * zh, chunk), :].astype(jnp.float32))

        for j in range(S):
            if OFF[j] == 0:
                gemm_and_first_send(j)
        for t in range(OFF[-1] + 5):
            for j in range(S):
                if OFF[j] == t + 1:
                    gemm_and_first_send(j)
            for j in range(S):
                st = t - OFF[j]
                if 0 <= st <= 2:
                    process_a(j, st)
                elif st == 3:
                    process_b1(j)
                elif st == 4:
                    process_b2(j)

        for j in range(S):
            for s in (1, 2):
                make_a_rdma(j, s).wait_send()
            make_b1_rdma(j).wait_send()
            make_b2_rdma(j).wait_send()

    return pl.pallas_call(
        body,
        out_shape=jax.ShapeDtypeStruct((chunk, f), jnp.float32),
        in_specs=[
            pl.BlockSpec(memory_space=pltpu.VMEM),
            pl.BlockSpec(memory_space=pltpu.VMEM),
        ],
        out_specs=pl.BlockSpec(memory_space=pltpu.VMEM),
        scratch_shapes=[
            pltpu.VMEM((m, d), jnp.bfloat16),
            pltpu.VMEM((NP, ar, f), jnp.bfloat16),
            pltpu.VMEM((S, 2, ar, fs), jnp.bfloat16),
            pltpu.VMEM((S, 3, ar, fs), jnp.bfloat16),
            pltpu.VMEM((S, ar, fs), jnp.bfloat16),
            pltpu.VMEM((S, 2 * chunk, fs), jnp.bfloat16),
            pltpu.VMEM((S, 2 * chunk, fs), jnp.bfloat16),
            pltpu.VMEM((S, 2 * chunk, fs), jnp.bfloat16),
            pltpu.VMEM((S, chunk, fs), jnp.bfloat16),
            pltpu.VMEM((S, chunk, fs), jnp.bfloat16),
            pltpu.SemaphoreType.DMA((S, 2)),
            pltpu.SemaphoreType.DMA((S, 3)),
            pltpu.SemaphoreType.DMA((S,)),
            pltpu.SemaphoreType.DMA((S,)),
            pltpu.SemaphoreType.DMA((S,)),
            pltpu.SemaphoreType.DMA((S,)),
        ],
        compiler_params=pltpu.CompilerParams(collective_id=0),
    )(x, dy)


# device time: 59363 ns/iter; 1.3823x vs baseline; 1.1542x over previous
import jax
import jax.numpy as jnp
from jax import lax
from jax.experimental import pallas as pl
from jax.experimental.pallas import tpu as pltpu

N_DEV = 16
NP = 4
S = 8
GRP = [0, 0, 0, 0, 1, 1, 2, 2]
OFF = [1, 1, 2, 2, 0, 0, 0, 0]
SNAKE = [(0, 0), (1, 0), (1, 1), (0, 1)]


def _block_tables():
    p_tab = [[NP * zb + qb for zb in range(NP)] for qb in range(NP)]
    xz_tab = []
    for u in range(NP):
        xu, zlu = SNAKE[u]
        xz_tab.append([4 * (2 * (w // 2) + zlu) + 2 * (w % 2) + (xu ^ (w % 2))
                       for w in range(NP)])
    yz_tab = []
    for v in range(NP):
        yv, zlv = SNAKE[v]
        yz_tab.append([4 * (2 * (w // 2) + zlv) + 2 * yv + ((w % 2) ^ yv)
                       for w in range(NP)])
    return [p_tab, xz_tab, yz_tab]


BLOCKS = _block_tables()


def kernel(x, dy):
    m, d = x.shape
    _, f = dy.shape
    chunk = d // N_DEV
    fs = f // S
    ar = NP * chunk

    def body(x_ref, dy_ref, out_ref,
             xb_ref, arr_ref, sa_ref, ca_ref, ra_ref, rb_ref,
             sb1_ref, cb1_ref, sb2_ref, cb2_ref,
             sa_send_sems, sa_recv_sems,
             b1_send_sems, b1_recv_sems, b2_send_sems, b2_recv_sems):
        my = lax.axis_index("i")
        z = lax.div(my, NP)
        q = lax.rem(my, NP)
        y = lax.div(q, 2)
        xc = lax.rem(q + y, 2)
        zl = lax.rem(z, 2)
        zh = lax.div(z, 2)
        plane_r = NP * z + lax.rem(q + 1, NP)
        plane_l = NP * z + lax.rem(q + NP - 1, NP)

        def xz_dev(un):
            zlu = lax.div(un, 2)
            xu = lax.rem(un + zlu, 2)
            return 4 * (2 * zh + zlu) + 2 * y + lax.rem(xu + y, 2)

        def yz_dev(vn):
            zlv = lax.div(vn, 2)
            yv = lax.rem(vn + zlv, 2)
            return 4 * (2 * zh + zlv) + 2 * yv + lax.rem(xc + yv, 2)

        u_g = [q,
               2 * zl + lax.rem(xc + zl, 2),
               2 * zl + lax.rem(y + zl, 2)]
        fwd_dev = [plane_r,
                   xz_dev(lax.rem(u_g[1] + 1, NP)),
                   yz_dev(lax.rem(u_g[2] + 1, NP))]
        bwd_dev = [plane_l,
                   xz_dev(lax.rem(u_g[1] + 3, NP)),
                   yz_dev(lax.rem(u_g[2] + 3, NP))]
        kb_g = [zl, y, xc]
        b1_dev = [my + NP * (1 - 2 * zl),
                  NP * z + 2 * (1 - y) + lax.rem(xc + 1 + y, 2),
                  NP * z + 2 * y + lax.rem(1 + xc + y, 2)]
        dev_b2 = my + 2 * NP * (1 - 2 * zh)

        barrier_sem = pltpu.get_barrier_semaphore()
        for nbr in (plane_l, plane_r, b1_dev[0], dev_b2):
            pl.semaphore_signal(
                barrier_sem, inc=1,
                device_id=(nbr,), device_id_type=pl.DeviceIdType.MESH,
            )
        pl.semaphore_wait(barrier_sem, 4)

        xb_ref[...] = x_ref[...].astype(jnp.bfloat16)

        def fcols(j):
            return slice(j * fs, (j + 1) * fs)

        def a_send_idx(j, s):
            u = u_g[GRP[j]]
            if j % 2 == 0:
                return lax.rem(u + 2 * NP - 1 - s, NP)
            return lax.rem(u + 1 + s, NP)

        def make_a_rdma(j, s):
            g = GRP[j]
            return pltpu.make_async_remote_copy(
                src_ref=sa_ref.at[j, s % 2],
                dst_ref=ca_ref.at[j, s],
                send_sem=sa_send_sems.at[j, s % 2],
                recv_sem=sa_recv_sems.at[j, s],
                device_id=(fwd_dev[g] if j % 2 == 0 else bwd_dev[g],),
                device_id_type=pl.DeviceIdType.MESH,
            )

        def make_b1_rdma(j):
            return pltpu.make_async_remote_copy(
                src_ref=sb1_ref.at[j], dst_ref=cb1_ref.at[j],
                send_sem=b1_send_sems.at[j], recv_sem=b1_recv_sems.at[j],
                device_id=(b1_dev[GRP[j]],),
                device_id_type=pl.DeviceIdType.MESH,
            )

        def make_b2_rdma(j):
            return pltpu.make_async_remote_copy(
                src_ref=sb2_ref.at[j], dst_ref=cb2_ref.at[j],
                send_sem=b2_send_sems.at[j], recv_sem=b2_recv_sems.at[j],
                device_id=(dev_b2,), device_id_type=pl.DeviceIdType.MESH,
            )

        def gemm_and_first_send(j):
            val = lax.dot_general(
                xb_ref[...], dy_ref[:, fcols(j)].astype(jnp.bfloat16),
                dimension_numbers=(((0,), (0,)), ((), ())),
                preferred_element_type=jnp.float32,
            ).astype(jnp.bfloat16)
            tab = BLOCKS[GRP[j]]
            for u in range(NP):
                for w in range(NP):
                    b = tab[u][w]
                    arr_ref[u, w * chunk:(w + 1) * chunk, fcols(j)] = (
                        val[b * chunk:(b + 1) * chunk, :])
            sa_ref[j, 0] = arr_ref[a_send_idx(j, 0), :, fcols(j)]
            make_a_rdma(j, 0).start()

        def process_a(j, s):
            make_a_rdma(j, s).wait_recv()
            a = a_send_idx(j, s + 1)
            acc = ca_ref[j, s] + arr_ref[a, :, fcols(j)]
            if s < 2:
                if s >= 1:
                    make_a_rdma(j, s - 1).wait_send()
                sa_ref[j, (s + 1) % 2] = acc
                make_a_rdma(j, s + 1).start()
            else:
                ra_ref[j] = acc
                lp = 1 - kb_g[GRP[j]]
                sb1_ref[j, 0:chunk] = ra_ref[j, pl.ds(chunk * lp, chunk), :]
                sb1_ref[j, chunk:2 * chunk] = (
                    ra_ref[j, pl.ds(chunk * (lp + 2), chunk), :])
                make_b1_rdma(j).start()

        def process_b1(j):
            make_b1_rdma(j).wait_recv()
            kb = kb_g[GRP[j]]
            rb_ref[j, 0:chunk] = (
                cb1_ref[j, 0:chunk] + ra_ref[j, pl.ds(chunk * kb, chunk), :])
            rb_ref[j, chunk:2 * chunk] = (
                cb1_ref[j, chunk:2 * chunk]
                + ra_ref[j, pl.ds(chunk * (kb + 2), chunk), :])
            sb2_ref[j] = rb_ref[j, pl.ds(chunk * (1 - zh), chunk), :]
            make_b2_rdma(j).start()

        def process_b2(j):
            make_b2_rdma(j).wait_recv()
            out_ref[:, fcols(j)] = (
                cb2_ref[j].astype(jnp.float32)
                + rb_ref[j, pl.ds(chunk * zh, chunk), :].astype(jnp.float32))

        for j in range(S):
            if OFF[j] == 0:
                gemm_and_first_send(j)
        for t in range(max(OFF) + 5):
            for j in range(S):
                if OFF[j] == t + 1:
                    gemm_and_first_send(j)
            for j in range(S):
                st = t - OFF[j]
                if 0 <= st <= 2:
                    process_a(j, st)
                elif st == 3:
                    process_b1(j)
                elif st == 4:
                    process_b2(j)

        for j in range(S):
            for s in (1, 2):
                make_a_rdma(j, s).wait_send()
            make_b1_rdma(j).wait_send()
            make_b2_rdma(j).wait_send()

    return pl.pallas_call(
        body,
        out_shape=jax.ShapeDtypeStruct((chunk, f), jnp.float32),
        in_specs=[
            pl.BlockSpec(memory_space=pltpu.VMEM),
            pl.BlockSpec(memory_space=pltpu.VMEM),
        ],
        out_specs=pl.BlockSpec(memory_space=pltpu.VMEM),
        scratch_shapes=[
            pltpu.VMEM((m, d), jnp.bfloat16),
            pltpu.VMEM((NP, ar, f), jnp.bfloat16),
            pltpu.VMEM((S, 2, ar, fs), jnp.bfloat16),
            pltpu.VMEM((S, 3, ar, fs), jnp.bfloat16),
            pltpu.VMEM((S, ar, fs), jnp.bfloat16),
            pltpu.VMEM((S, 2 * chunk, fs), jnp.bfloat16),
            pltpu.VMEM((S, 2 * chunk, fs), jnp.bfloat16),
            pltpu.VMEM((S, 2 * chunk, fs), jnp.bfloat16),
            pltpu.VMEM((S, chunk, fs), jnp.bfloat16),
            pltpu.VMEM((S, chunk, fs), jnp.bfloat16),
            pltpu.SemaphoreType.DMA((S, 2)),
            pltpu.SemaphoreType.DMA((S, 3)),
            pltpu.SemaphoreType.DMA((S,)),
            pltpu.SemaphoreType.DMA((S,)),
            pltpu.SemaphoreType.DMA((S,)),
            pltpu.SemaphoreType.DMA((S,)),
        ],
        compiler_params=pltpu.CompilerParams(collective_id=0),
    )(x, dy)


# device time: 57607 ns/iter; 1.4245x vs baseline; 1.0305x over previous
import jax
import jax.numpy as jnp
from jax import lax
from jax.experimental import pallas as pl
from jax.experimental.pallas import tpu as pltpu

N_DEV = 16
NP = 4
S = 8
GRP = [0, 0, 0, 0, 1, 1, 2, 2]
OFF = [1, 1, 2, 2, 0, 0, 1, 1]
SNAKE = [(0, 0), (1, 0), (1, 1), (0, 1)]


def _block_tables():
    p_tab = [[NP * zb + qb for zb in range(NP)] for qb in range(NP)]
    xz_tab = []
    for u in range(NP):
        xu, zlu = SNAKE[u]
        xz_tab.append([4 * (2 * (w // 2) + zlu) + 2 * (w % 2) + (xu ^ (w % 2))
                       for w in range(NP)])
    yz_tab = []
    for v in range(NP):
        yv, zlv = SNAKE[v]
        yz_tab.append([4 * (2 * (w // 2) + zlv) + 2 * yv + ((w % 2) ^ yv)
                       for w in range(NP)])
    return [p_tab, xz_tab, yz_tab]


BLOCKS = _block_tables()


def kernel(x, dy):
    m, d = x.shape
    _, f = dy.shape
    chunk = d // N_DEV
    fs = f // S
    ar = NP * chunk

    def body(x_ref, dy_ref, out_ref,
             xb_ref, arr_ref, sa_ref, ca_ref, ra_ref, rb_ref,
             sb1_ref, cb1_ref, sb2_ref, cb2_ref,
             sa_send_sems, sa_recv_sems,
             b1_send_sems, b1_recv_sems, b2_send_sems, b2_recv_sems):
        my = lax.axis_index("i")
        z = lax.div(my, NP)
        q = lax.rem(my, NP)
        y = lax.div(q, 2)
        xc = lax.rem(q + y, 2)
        zl = lax.rem(z, 2)
        zh = lax.div(z, 2)
        plane_r = NP * z + lax.rem(q + 1, NP)
        plane_l = NP * z + lax.rem(q + NP - 1, NP)

        def xz_dev(un):
            zlu = lax.div(un, 2)
            xu = lax.rem(un + zlu, 2)
            return 4 * (2 * zh + zlu) + 2 * y + lax.rem(xu + y, 2)

        def yz_dev(vn):
            zlv = lax.div(vn, 2)
            yv = lax.rem(vn + zlv, 2)
            return 4 * (2 * zh + zlv) + 2 * yv + lax.rem(xc + yv, 2)

        u_g = [q,
               2 * zl + lax.rem(xc + zl, 2),
               2 * zl + lax.rem(y + zl, 2)]
        fwd_dev = [plane_r,
                   xz_dev(lax.rem(u_g[1] + 1, NP)),
                   yz_dev(lax.rem(u_g[2] + 1, NP))]
        bwd_dev = [plane_l,
                   xz_dev(lax.rem(u_g[1] + 3, NP)),
                   yz_dev(lax.rem(u_g[2] + 3, NP))]
        kb_g = [zl, y, xc]
        b1_dev = [my + NP * (1 - 2 * zl),
                  NP * z + 2 * (1 - y) + lax.rem(xc + 1 + y, 2),
                  NP * z + 2 * y + lax.rem(1 + xc + y, 2)]
        dev_b2 = my + 2 * NP * (1 - 2 * zh)

        barrier_sem = pltpu.get_barrier_semaphore()
        for nbr in (plane_l, plane_r, b1_dev[0], dev_b2):
            pl.semaphore_signal(
                barrier_sem, inc=1,
                device_id=(nbr,), device_id_type=pl.DeviceIdType.MESH,
            )
        pl.semaphore_wait(barrier_sem, 4)

        xb_ref[...] = x_ref[...].astype(jnp.bfloat16)

        def fcols(j):
            return slice(j * fs, (j + 1) * fs)

        def a_send_idx(j, s):
            u = u_g[GRP[j]]
            if j % 2 == 0:
                return lax.rem(u + 2 * NP - 1 - s, NP)
            return lax.rem(u + 1 + s, NP)

        def make_a_rdma(j, s):
            g = GRP[j]
            return pltpu.make_async_remote_copy(
                src_ref=sa_ref.at[j, s % 2],
                dst_ref=ca_ref.at[j, s],
                send_sem=sa_send_sems.at[j, s % 2],
                recv_sem=sa_recv_sems.at[j, s],
                device_id=(fwd_dev[g] if j % 2 == 0 else bwd_dev[g],),
                device_id_type=pl.DeviceIdType.MESH,
            )

        def make_b1_rdma(j):
            return pltpu.make_async_remote_copy(
                src_ref=sb1_ref.at[j], dst_ref=cb1_ref.at[j],
                send_sem=b1_send_sems.at[j], recv_sem=b1_recv_sems.at[j],
                device_id=(b1_dev[GRP[j]],),
                device_id_type=pl.DeviceIdType.MESH,
            )

        def make_b2_rdma(j):
            return pltpu.make_async_remote_copy(
                src_ref=sb2_ref.at[j], dst_ref=cb2_ref.at[j],
                send_sem=b2_send_sems.at[j], recv_sem=b2_recv_sems.at[j],
                device_id=(dev_b2,), device_id_type=pl.DeviceIdType.MESH,
            )

        def gemm_and_first_send(j):
            val = lax.dot_general(
                xb_ref[...], dy_ref[:, fcols(j)].astype(jnp.bfloat16),
                dimension_numbers=(((0,), (0,)), ((), ())),
                preferred_element_type=jnp.float32,
            ).astype(jnp.bfloat16)
            tab = BLOCKS[GRP[j]]
            for u in range(NP):
                for w in range(NP):
                    b = tab[u][w]
                    arr_ref[u, w * chunk:(w + 1) * chunk, fcols(j)] = (
                        val[b * chunk:(b + 1) * chunk, :])
            sa_ref[j, 0] = arr_ref[a_send_idx(j, 0), :, fcols(j)]
            make_a_rdma(j, 0).start()

        def process_a(j, s):
            make_a_rdma(j, s).wait_recv()
            a = a_send_idx(j, s + 1)
            acc = ca_ref[j, s] + arr_ref[a, :, fcols(j)]
            if s < 2:
                if s >= 1:
                    make_a_rdma(j, s - 1).wait_send()
                sa_ref[j, (s + 1) % 2] = acc
                make_a_rdma(j, s + 1).start()
            else:
                ra_ref[j] = acc
                lp = 1 - kb_g[GRP[j]]
                sb1_ref[j, 0:chunk] = ra_ref[j, pl.ds(chunk * lp, chunk), :]
                sb1_ref[j, chunk:2 * chunk] = (
                    ra_ref[j, pl.ds(chunk * (lp + 2), chunk), :])
                make_b1_rdma(j).start()

        def process_b1(j):
            make_b1_rdma(j).wait_recv()
            kb = kb_g[GRP[j]]
            rb_ref[j, 0:chunk] = (
                cb1_ref[j, 0:chunk] + ra_ref[j, pl.ds(chunk * kb, chunk), :])
            rb_ref[j, chunk:2 * chunk] = (
                cb1_ref[j, chunk:2 * chunk]
                + ra_ref[j, pl.ds(chunk * (kb + 2), chunk), :])
            sb2_ref[j] = rb_ref[j, pl.ds(chunk * (1 - zh), chunk), :]
            make_b2_rdma(j).start()

        def process_b2(j):
            make_b2_rdma(j).wait_recv()
            out_ref[:, fcols(j)] = (
                cb2_ref[j].astype(jnp.float32)
                + rb_ref[j, pl.ds(chunk * zh, chunk), :].astype(jnp.float32))

        for j in range(S):
            if OFF[j] == 0:
                gemm_and_first_send(j)
        for t in range(max(OFF) + 5):
            for j in range(S):
                if OFF[j] == t + 1:
                    gemm_and_first_send(j)
            for j in range(S):
                st = t - OFF[j]
                if 0 <= st <= 2:
                    process_a(j, st)
                elif st == 3:
                    process_b1(j)
                elif st == 4:
                    process_b2(j)

        for j in range(S):
            for s in (1, 2):
                make_a_rdma(j, s).wait_send()
            make_b1_rdma(j).wait_send()
            make_b2_rdma(j).wait_send()

    return pl.pallas_call(
        body,
        out_shape=jax.ShapeDtypeStruct((chunk, f), jnp.float32),
        in_specs=[
            pl.BlockSpec(memory_space=pltpu.VMEM),
            pl.BlockSpec(memory_space=pltpu.VMEM),
        ],
        out_specs=pl.BlockSpec(memory_space=pltpu.VMEM),
        scratch_shapes=[
            pltpu.VMEM((m, d), jnp.bfloat16),
            pltpu.VMEM((NP, ar, f), jnp.bfloat16),
            pltpu.VMEM((S, 2, ar, fs), jnp.bfloat16),
            pltpu.VMEM((S, 3, ar, fs), jnp.bfloat16),
            pltpu.VMEM((S, ar, fs), jnp.bfloat16),
            pltpu.VMEM((S, 2 * chunk, fs), jnp.bfloat16),
            pltpu.VMEM((S, 2 * chunk, fs), jnp.bfloat16),
            pltpu.VMEM((S, 2 * chunk, fs), jnp.bfloat16),
            pltpu.VMEM((S, chunk, fs), jnp.bfloat16),
            pltpu.VMEM((S, chunk, fs), jnp.bfloat16),
            pltpu.SemaphoreType.DMA((S, 2)),
            pltpu.SemaphoreType.DMA((S, 3)),
            pltpu.SemaphoreType.DMA((S,)),
            pltpu.SemaphoreType.DMA((S,)),
            pltpu.SemaphoreType.DMA((S,)),
            pltpu.SemaphoreType.DMA((S,)),
        ],
        compiler_params=pltpu.CompilerParams(collective_id=0),
    )(x, dy)
